# baseline (speedup 1.0000x reference)
import functools

import jax
import jax.numpy as jnp
from jax import lax
from jax.experimental import pallas as pl
from jax.experimental.pallas import tpu as pltpu
from jax.experimental.pallas import tpu_sc as plsc

_BLK = 256
_HCHUNK = 2048


def _mlp_body(be_ref, nu_ref, x_ref, w1_ref, b1_ref, w2_ref, b2_ref, wt_ref,
              out_ref, h_ref, *, hidden, hchunk):
    j = pl.program_id(0)

    @pl.when(j < nu_ref[0])
    def _():
        xb = x_ref[...].astype(jnp.bfloat16)
        for hk in range(hidden // hchunk):
            sl = slice(hk * hchunk, (hk + 1) * hchunk)
            h = jnp.dot(xb, w1_ref[0, :, sl],
                        preferred_element_type=jnp.float32)
            h_ref[:, sl] = jnp.maximum(
                h + b1_ref[0, 0, sl][None, :], 0.0).astype(jnp.bfloat16)
        y = jnp.dot(h_ref[...], w2_ref[0],
                    preferred_element_type=jnp.float32)
        out_ref[...] = (y + b2_ref[0, 0][None, :]) * wt_ref[...]


def _grouped_mlp(block_expert, n_used, x_pad, W1, b1, W2, b2, w_pad):
    P, D = x_pad.shape
    E, _, H = W1.shape
    J = P // _BLK
    grid_spec = pltpu.PrefetchScalarGridSpec(
        num_scalar_prefetch=2,
        grid=(J,),
        in_specs=[
            pl.BlockSpec((_BLK, D), lambda j, be, nu: (j, 0)),
            pl.BlockSpec((1, D, H), lambda j, be, nu: (be[j], 0, 0)),
            pl.BlockSpec((1, 1, H), lambda j, be, nu: (be[j], 0, 0)),
            pl.BlockSpec((1, H, D), lambda j, be, nu: (be[j], 0, 0)),
            pl.BlockSpec((1, 1, D), lambda j, be, nu: (be[j], 0, 0)),
            pl.BlockSpec((_BLK, 1), lambda j, be, nu: (j, 0)),
        ],
        out_specs=pl.BlockSpec((_BLK, D), lambda j, be, nu: (j, 0)),
        scratch_shapes=[pltpu.VMEM((_BLK, H), jnp.bfloat16)],
    )
    body = functools.partial(_mlp_body, hidden=H, hchunk=min(_HCHUNK, H))
    return pl.pallas_call(
        body,
        grid_spec=grid_spec,
        out_shape=jax.ShapeDtypeStruct((P, D), jnp.float32),
        compiler_params=pltpu.CompilerParams(
            dimension_semantics=("arbitrary",)),
    )(block_expert, n_used, x_pad, W1.astype(jnp.bfloat16), b1[:, None, :],
      W2.astype(jnp.bfloat16), b2[:, None, :], w_pad)


def _sc_dispatch_gather(x, pad_tok):
    T, D = x.shape
    P = pad_tok.shape[0]
    info = plsc.get_sparse_core_info()
    nw = info.num_cores * info.num_subcores
    per_w = P // nw
    ch = 32
    nch = per_w // ch
    mesh = plsc.VectorSubcoreMesh(core_axis_name="c", subcore_axis_name="s")

    @functools.partial(
        pl.kernel, mesh=mesh,
        out_type=jax.ShapeDtypeStruct((P, D), x.dtype),
        scratch_types=[
            pltpu.VMEM((per_w,), jnp.int32),
            pltpu.VMEM((2, ch, D), x.dtype),
            pltpu.SemaphoreType.DMA,
            pltpu.SemaphoreType.DMA,
        ],
    )
    def k(x_hbm, idx_hbm, out_hbm, idx_v, rows_v, s0, s1):
        wid = lax.axis_index("s") * info.num_cores + lax.axis_index("c")
        base = wid * per_w
        pltpu.sync_copy(idx_hbm.at[pl.ds(base, per_w)], idx_v)
        sems = (s0, s1)

        def start(c):
            return pltpu.async_copy(
                x_hbm.at[idx_v.at[pl.ds(c * ch, ch)]],
                rows_v.at[c % 2], sems[c % 2])

        cps = {0: start(0)}
        for c in range(1, nch):
            cps[c] = start(c)
            cps[c - 1].wait()
            pltpu.sync_copy(rows_v.at[(c - 1) % 2],
                            out_hbm.at[pl.ds(base + (c - 1) * ch, ch)])
        cps[nch - 1].wait()
        pltpu.sync_copy(rows_v.at[(nch - 1) % 2],
                        out_hbm.at[pl.ds(base + (nch - 1) * ch, ch)])

    return k(x, pad_tok)


def _sc_combine(y_pad, pk0, pk1):
    P, D = y_pad.shape
    T = pk0.shape[0]
    info = plsc.get_sparse_core_info()
    nw = info.num_cores * info.num_subcores
    per_w = T // nw
    ch = 32
    lanes = 32 if y_pad.dtype == jnp.bfloat16 else 16
    mesh = plsc.VectorSubcoreMesh(core_axis_name="c", subcore_axis_name="s")

    @functools.partial(
        pl.kernel, mesh=mesh,
        out_type=jax.ShapeDtypeStruct((T, D), y_pad.dtype),
        scratch_types=[
            pltpu.VMEM((per_w,), jnp.int32),
            pltpu.VMEM((per_w,), jnp.int32),
            pltpu.VMEM((ch, D), y_pad.dtype),
            pltpu.VMEM((ch, D), y_pad.dtype),
            pltpu.SemaphoreType.DMA,
            pltpu.SemaphoreType.DMA,
        ],
    )
    def k(y_hbm, pk0_hbm, pk1_hbm, out_hbm, i0_v, i1_v, b0_v, b1_v, s0, s1):
        wid = lax.axis_index("s") * info.num_cores + lax.axis_index("c")
        base = wid * per_w
        pltpu.sync_copy(pk0_hbm.at[pl.ds(base, per_w)], i0_v)
        pltpu.sync_copy(pk1_hbm.at[pl.ds(base, per_w)], i1_v)

        def chunk(c, carry):
            cp0 = pltpu.async_copy(
                y_hbm.at[i0_v.at[pl.ds(c * ch, ch)]], b0_v, s0)
            cp1 = pltpu.async_copy(
                y_hbm.at[i1_v.at[pl.ds(c * ch, ch)]], b1_v, s1)
            cp0.wait()
            cp1.wait()

            def row(r, rc):
                for cc in range(D // lanes):
                    sl = pl.ds(cc * lanes, lanes)
                    b0_v[r, sl] = b0_v[r, sl] + b1_v[r, sl]
                return rc

            lax.fori_loop(0, ch, row, 0)
            pltpu.sync_copy(b0_v, out_hbm.at[pl.ds(base + c * ch, ch)])
            return carry

        lax.fori_loop(0, per_w // ch, chunk, 0)

    return k(y_pad, pk0, pk1)


def kernel(expert_input, gate_w, gate_b, W1, b1, W2, b2):
    T, D = expert_input.shape
    E, _, H = W1.shape
    K = min(2, E)
    B = _BLK
    P = T * K + E * B

    logits = expert_input @ gate_w + gate_b
    eids = jnp.arange(E, dtype=jnp.int32)
    m1 = jnp.max(logits, axis=1)
    i1 = jnp.argmax(logits, axis=1).astype(jnp.int32)
    masked = jnp.where(i1[:, None] == eids[None, :], -jnp.inf, logits)
    m2 = jnp.max(masked, axis=1)
    i2 = jnp.argmax(masked, axis=1).astype(jnp.int32)
    w1g = jax.nn.sigmoid(m1 - m2)
    topi = jnp.stack([i1, i2], axis=1)
    topw = jnp.stack([w1g, 1.0 - w1g], axis=1)

    e_flat = topi.reshape(-1)
    perm = jnp.argsort(e_flat, stable=True).astype(jnp.int32)
    e_sorted = e_flat[perm]
    cnt = jnp.bincount(e_flat, length=E)
    aligned = ((cnt + B - 1) // B) * B
    pstart = (jnp.cumsum(aligned) - aligned).astype(jnp.int32)
    offs = (jnp.cumsum(cnt) - cnt).astype(jnp.int32)
    i = jnp.arange(T * K, dtype=jnp.int32)
    pp_sorted = pstart[e_sorted] + (i - offs[e_sorted])
    pad_tok = jnp.zeros((P,), jnp.int32).at[pp_sorted].set(
        (perm // K).astype(jnp.int32))
    w_pad = jnp.zeros((P, 1), jnp.float32).at[pp_sorted, 0].set(
        topw.reshape(-1)[perm])
    pp_flat = jnp.zeros((T * K,), jnp.int32).at[perm].set(pp_sorted)
    blk_start = pstart // B
    block_expert = jnp.clip(
        jnp.searchsorted(blk_start, jnp.arange(P // B), side="right") - 1,
        0, E - 1).astype(jnp.int32)
    n_used = (jnp.sum(aligned) // B).astype(jnp.int32)[None]

    x_pad = _sc_dispatch_gather(expert_input, pad_tok)
    y_pad = _grouped_mlp(block_expert, n_used, x_pad, W1, b1, W2, b2, w_pad)
    pk = pp_flat.reshape(T, K)
    out = _sc_combine(y_pad, pk[:, 0], pk[:, 1])
    return out


# speedup vs baseline: 1.0037x; 1.0037x over previous
import functools

import jax
import jax.numpy as jnp
from jax import lax
from jax.experimental import pallas as pl
from jax.experimental.pallas import tpu as pltpu
from jax.experimental.pallas import tpu_sc as plsc

_BLK = 256
_HCHUNK = 2048


def _mlp_body(be_ref, nu_ref, x_ref, w1_ref, b1_ref, w2_ref, b2_ref, wt_ref,
              out_ref, h_ref, *, hidden, hchunk):
    j = pl.program_id(0)

    @pl.when(j < nu_ref[0])
    def _():
        xb = x_ref[...].astype(jnp.bfloat16)
        for hk in range(hidden // hchunk):
            sl = slice(hk * hchunk, (hk + 1) * hchunk)
            h = jnp.dot(xb, w1_ref[0, :, sl],
                        preferred_element_type=jnp.float32)
            h_ref[:, sl] = jnp.maximum(
                h + b1_ref[0, 0, sl][None, :], 0.0).astype(jnp.bfloat16)
        y = jnp.dot(h_ref[...], w2_ref[0],
                    preferred_element_type=jnp.float32)
        out_ref[...] = (y + b2_ref[0, 0][None, :]) * wt_ref[...]


def _grouped_mlp(block_expert, n_used, x_pad, W1, b1, W2, b2, w_pad):
    P, D = x_pad.shape
    E, _, H = W1.shape
    J = P // _BLK
    grid_spec = pltpu.PrefetchScalarGridSpec(
        num_scalar_prefetch=2,
        grid=(J,),
        in_specs=[
            pl.BlockSpec((_BLK, D), lambda j, be, nu: (j, 0)),
            pl.BlockSpec((1, D, H), lambda j, be, nu: (be[j], 0, 0)),
            pl.BlockSpec((1, 1, H), lambda j, be, nu: (be[j], 0, 0)),
            pl.BlockSpec((1, H, D), lambda j, be, nu: (be[j], 0, 0)),
            pl.BlockSpec((1, 1, D), lambda j, be, nu: (be[j], 0, 0)),
            pl.BlockSpec((_BLK, 1), lambda j, be, nu: (j, 0)),
        ],
        out_specs=pl.BlockSpec((_BLK, D), lambda j, be, nu: (j, 0)),
        scratch_shapes=[pltpu.VMEM((_BLK, H), jnp.bfloat16)],
    )
    body = functools.partial(_mlp_body, hidden=H, hchunk=min(_HCHUNK, H))
    return pl.pallas_call(
        body,
        grid_spec=grid_spec,
        out_shape=jax.ShapeDtypeStruct((P, D), jnp.float32),
        compiler_params=pltpu.CompilerParams(
            dimension_semantics=("arbitrary",)),
    )(block_expert, n_used, x_pad, W1.astype(jnp.bfloat16), b1[:, None, :],
      W2.astype(jnp.bfloat16), b2[:, None, :], w_pad)


def _sc_dispatch_gather(x, pad_tok):
    T, D = x.shape
    P = pad_tok.shape[0]
    info = plsc.get_sparse_core_info()
    nw = info.num_cores * info.num_subcores
    per_w = P // nw
    ch = 32
    nch = per_w // ch
    mesh = plsc.VectorSubcoreMesh(core_axis_name="c", subcore_axis_name="s")

    @functools.partial(
        pl.kernel, mesh=mesh,
        out_type=jax.ShapeDtypeStruct((P, D), x.dtype),
        scratch_types=[
            pltpu.VMEM((per_w,), jnp.int32),
            pltpu.VMEM((2, ch, D), x.dtype),
            pltpu.SemaphoreType.DMA,
            pltpu.SemaphoreType.DMA,
        ],
    )
    def k(x_hbm, idx_hbm, out_hbm, idx_v, rows_v, s0, s1):
        wid = lax.axis_index("s") * info.num_cores + lax.axis_index("c")
        base = wid * per_w
        pltpu.sync_copy(idx_hbm.at[pl.ds(base, per_w)], idx_v)
        sems = (s0, s1)

        def start(c):
            return pltpu.async_copy(
                x_hbm.at[idx_v.at[pl.ds(c * ch, ch)]],
                rows_v.at[c % 2], sems[c % 2])

        cps = {0: start(0)}
        for c in range(1, nch):
            cps[c] = start(c)
            cps[c - 1].wait()
            pltpu.sync_copy(rows_v.at[(c - 1) % 2],
                            out_hbm.at[pl.ds(base + (c - 1) * ch, ch)])
        cps[nch - 1].wait()
        pltpu.sync_copy(rows_v.at[(nch - 1) % 2],
                        out_hbm.at[pl.ds(base + (nch - 1) * ch, ch)])

    return k(x, pad_tok)


def _sc_combine(y_pad, pk0, pk1):
    P, D = y_pad.shape
    T = pk0.shape[0]
    info = plsc.get_sparse_core_info()
    nw = info.num_cores * info.num_subcores
    per_w = T // nw
    ch = 16
    nch = per_w // ch
    lanes = 32 if y_pad.dtype == jnp.bfloat16 else 16
    mesh = plsc.VectorSubcoreMesh(core_axis_name="c", subcore_axis_name="s")

    @functools.partial(
        pl.kernel, mesh=mesh,
        out_type=jax.ShapeDtypeStruct((T, D), y_pad.dtype),
        scratch_types=[
            pltpu.VMEM((per_w,), jnp.int32),
            pltpu.VMEM((per_w,), jnp.int32),
            pltpu.VMEM((2, ch, D), y_pad.dtype),
            pltpu.VMEM((2, ch, D), y_pad.dtype),
            pltpu.SemaphoreType.DMA,
            pltpu.SemaphoreType.DMA,
        ],
    )
    def k(y_hbm, pk0_hbm, pk1_hbm, out_hbm, i0_v, i1_v, b0_v, b1_v, s0, s1):
        wid = lax.axis_index("s") * info.num_cores + lax.axis_index("c")
        base = wid * per_w
        pltpu.sync_copy(pk0_hbm.at[pl.ds(base, per_w)], i0_v)
        pltpu.sync_copy(pk1_hbm.at[pl.ds(base, per_w)], i1_v)
        sems = (s0, s1)

        def start(c):
            slot = c % 2
            cp0 = pltpu.async_copy(
                y_hbm.at[i0_v.at[pl.ds(c * ch, ch)]], b0_v.at[slot],
                sems[slot])
            cp1 = pltpu.async_copy(
                y_hbm.at[i1_v.at[pl.ds(c * ch, ch)]], b1_v.at[slot],
                sems[slot])
            return cp0, cp1

        cps = {0: start(0)}
        for c in range(nch):
            if c + 1 < nch:
                cps[c + 1] = start(c + 1)
            cp0, cp1 = cps.pop(c)
            cp0.wait()
            cp1.wait()
            slot = c % 2

            def row(r, rc, slot=slot):
                for cc in range(D // lanes):
                    sl = pl.ds(cc * lanes, lanes)
                    b0_v[slot, r, sl] = b0_v[slot, r, sl] + b1_v[slot, r, sl]
                return rc

            lax.fori_loop(0, ch, row, 0)
            pltpu.sync_copy(b0_v.at[slot],
                            out_hbm.at[pl.ds(base + c * ch, ch)])

    return k(y_pad, pk0, pk1)


def kernel(expert_input, gate_w, gate_b, W1, b1, W2, b2):
    T, D = expert_input.shape
    E, _, H = W1.shape
    K = min(2, E)
    B = _BLK
    P = T * K + E * B

    logits = expert_input @ gate_w + gate_b
    eids = jnp.arange(E, dtype=jnp.int32)
    m1 = jnp.max(logits, axis=1)
    i1 = jnp.argmax(logits, axis=1).astype(jnp.int32)
    masked = jnp.where(i1[:, None] == eids[None, :], -jnp.inf, logits)
    m2 = jnp.max(masked, axis=1)
    i2 = jnp.argmax(masked, axis=1).astype(jnp.int32)
    w1g = jax.nn.sigmoid(m1 - m2)
    topi = jnp.stack([i1, i2], axis=1)
    topw = jnp.stack([w1g, 1.0 - w1g], axis=1)

    e_flat = topi.reshape(-1)
    perm = jnp.argsort(e_flat, stable=True).astype(jnp.int32)
    e_sorted = e_flat[perm]
    cnt = jnp.bincount(e_flat, length=E)
    aligned = ((cnt + B - 1) // B) * B
    pstart = (jnp.cumsum(aligned) - aligned).astype(jnp.int32)
    offs = (jnp.cumsum(cnt) - cnt).astype(jnp.int32)
    i = jnp.arange(T * K, dtype=jnp.int32)
    pp_sorted = pstart[e_sorted] + (i - offs[e_sorted])
    pad_tok = jnp.zeros((P,), jnp.int32).at[pp_sorted].set(
        (perm // K).astype(jnp.int32))
    w_pad = jnp.zeros((P, 1), jnp.float32).at[pp_sorted, 0].set(
        topw.reshape(-1)[perm])
    pp_flat = jnp.zeros((T * K,), jnp.int32).at[perm].set(pp_sorted)
    blk_start = pstart // B
    block_expert = jnp.clip(
        jnp.searchsorted(blk_start, jnp.arange(P // B), side="right") - 1,
        0, E - 1).astype(jnp.int32)
    n_used = (jnp.sum(aligned) // B).astype(jnp.int32)[None]

    x_pad = _sc_dispatch_gather(expert_input, pad_tok)
    y_pad = _grouped_mlp(block_expert, n_used, x_pad, W1, b1, W2, b2, w_pad)
    pk = pp_flat.reshape(T, K)
    out = _sc_combine(y_pad, pk[:, 0], pk[:, 1])
    return out
